# baseline (device time: 198596 ns/iter reference)
import jax
import jax.numpy as jnp
from jax import lax
from jax.experimental import pallas as pl
from jax.experimental.pallas import tpu as pltpu


def kernel(x, W, labels):
    T, D = x.shape
    _, V_loc = W.shape
    BV = 512
    NV = V_loc // BV

    def cast_body(x_ref, o_ref):
        o_ref[...] = x_ref[...].astype(jnp.bfloat16)

    x_bf = pl.pallas_call(
        cast_body,
        grid=(4,),
        in_specs=[pl.BlockSpec((T // 4, D), lambda i: (i, 0))],
        out_specs=pl.BlockSpec((T // 4, D), lambda i: (i, 0)),
        out_shape=jax.ShapeDtypeStruct((T, D), jnp.bfloat16),
    )(x)

    labels2d = labels.reshape(T, 1)

    def body(xbf_ref, w_ref, lab_ref, out_ref, acc_ref, recv_ref,
             send_sem, recv_sem):
        j = pl.program_id(0)
        my_x = lax.axis_index("x")
        my_y = lax.axis_index("y")
        my_z = lax.axis_index("z")

        @pl.when(j == 0)
        def _init():
            acc_ref[...] = jnp.zeros(acc_ref.shape, acc_ref.dtype)

        logits = jnp.dot(
            xbf_ref[...],
            w_ref[...].astype(jnp.bfloat16),
            preferred_element_type=jnp.float32,
        )
        acc_ref[0, :, :] += jnp.sum(jnp.exp(logits), axis=1, keepdims=True)
        base = my_x * V_loc + j * BV
        ids = base + lax.broadcasted_iota(jnp.int32, (T, BV), 1)
        acc_ref[1, :, :] += jnp.sum(
            jnp.where(ids == lab_ref[...], logits, 0.0),
            axis=1, keepdims=True,
        )

        @pl.when(j == NV - 1)
        def _exchange():
            partner = (1 - my_x, my_y, my_z)
            barrier = pltpu.get_barrier_semaphore()
            pl.semaphore_signal(
                barrier, inc=1, device_id=partner,
                device_id_type=pl.DeviceIdType.MESH,
            )
            pl.semaphore_wait(barrier, 1)

            rdma = pltpu.make_async_remote_copy(
                src_ref=acc_ref,
                dst_ref=recv_ref,
                send_sem=send_sem,
                recv_sem=recv_sem,
                device_id=partner,
                device_id_type=pl.DeviceIdType.MESH,
            )
            rdma.start()
            rdma.wait()

            s_tot = acc_ref[0, :, :] + recv_ref[0, :, :]
            ll_tot = acc_ref[1, :, :] + recv_ref[1, :, :]
            out_ref[...] = jnp.log(s_tot) - ll_tot

    nll2d = pl.pallas_call(
        body,
        grid=(NV,),
        in_specs=[
            pl.BlockSpec(memory_space=pltpu.VMEM),
            pl.BlockSpec((D, BV), lambda j: (0, j)),
            pl.BlockSpec(memory_space=pltpu.VMEM),
        ],
        out_specs=pl.BlockSpec((T, 1), lambda j: (0, 0)),
        out_shape=jax.ShapeDtypeStruct((T, 1), jnp.float32),
        scratch_shapes=[
            pltpu.VMEM((2, T, 1), jnp.float32),
            pltpu.VMEM((2, T, 1), jnp.float32),
            pltpu.SemaphoreType.DMA,
            pltpu.SemaphoreType.DMA,
        ],
        compiler_params=pltpu.CompilerParams(
            dimension_semantics=("arbitrary",),
            collective_id=0,
        ),
    )(x_bf, W, labels2d)

    return nll2d.reshape(T)
